# Initial kernel scaffold; baseline (speedup 1.0000x reference)
import jax
import jax.numpy as jnp
from jax.experimental import pallas as pl


def kernel(x, edge_index, W1, b1, W2, b2, W3, b3):
    raise NotImplementedError("write your pallas kernel here")


# speedup vs baseline: 2.7698x; 2.7698x over previous
import jax
import jax.numpy as jnp
from jax import lax

from jax.experimental import pallas as pl
from jax.experimental.pallas import tpu as pltpu
from jax.experimental.pallas import tpu_sc as plsc

N = 10000
D = 256
DH = D // 2
NC = 2
NS = 16
NPTA = 640
NPAD = NS * NPTA
NZC = 128

_MESH = plsc.VectorSubcoreMesh(
    core_axis_name="c", subcore_axis_name="s", num_cores=NC, num_subcores=NS)


def _scat_body(table_hbm, src_hbm, dst_hbm, out_hbm,
               idx_v, dst_v, rows_v, zb_v, acc, gsem):
  c = lax.axis_index("c")
  s = lax.axis_index("s")

  def _zero(i, _):
    for k in range(DH // 16):
      zb_v[i, pl.ds(k * 16, 16)] = jnp.zeros((16,), jnp.float32)
    return 0
  lax.fori_loop(0, NZC, _zero, 0)
  for ks in range(NS):
    @pl.when(s == ks)
    def _():
      for k in range(NPTA // NZC):
        pltpu.sync_copy(zb_v, acc.at[pl.ds(ks * NPTA + k * NZC, NZC)])
  plsc.subcore_barrier()

  off = c * N
  nslab = src_hbm.shape[1]
  slab = src_hbm.shape[2]

  def _slab(g, _):
    pltpu.sync_copy(src_hbm.at[s].at[g], idx_v)
    pltpu.sync_copy(dst_hbm.at[s].at[g], dst_v)

    def _shift(r, _):
      for k in range(idx_v.shape[1] // 16):
        idx_v[r, pl.ds(k * 16, 16)] = idx_v[r, pl.ds(k * 16, 16)] + off
      return 0
    lax.fori_loop(0, slab, _shift, 0)

    def _chunk(j, _):
      pltpu.async_copy(table_hbm.at[idx_v.at[j]], rows_v, gsem).wait()
      pltpu.sync_copy(rows_v, acc.at[dst_v.at[j]], add=True)
      return 0
    lax.fori_loop(0, slab, _chunk, 0)
    return 0
  lax.fori_loop(0, nslab, _slab, 0)
  plsc.subcore_barrier()

  nwb = NPTA // NZC
  for ks in range(NS):
    @pl.when(s == ks)
    def _():
      for k in range(nwb):
        pltpu.sync_copy(acc.at[pl.ds(ks * NPTA + k * NZC, NZC)], zb_v)
        pltpu.sync_copy(zb_v, out_hbm.at[c * (NS * nwb) + ks * nwb + k])


def _scat_call(table, src_r, dst_r, slab, ke):
  nwb = NPTA // NZC
  fn = pl.kernel(
      _scat_body,
      out_type=jax.ShapeDtypeStruct((NC * NS * nwb, NZC, DH), jnp.float32),
      mesh=_MESH,
      scratch_types=[
          pltpu.VMEM((slab, ke), jnp.int32),
          pltpu.VMEM((slab, ke), jnp.int32),
          pltpu.VMEM((ke, DH), jnp.float32),
          pltpu.VMEM((NZC, DH), jnp.float32),
          pltpu.VMEM_SHARED((NPAD, DH), jnp.float32),
          pltpu.SemaphoreType.DMA,
      ],
  )
  return fn(table, src_r, dst_r)


def _dinv_body(deg_ref, di_ref):
  di_ref[...] = lax.rsqrt(jnp.maximum(deg_ref[...] + 1.0, 1.0))


def _dinv_call(deg):
  return pl.pallas_call(
      _dinv_body,
      out_shape=jax.ShapeDtypeStruct((N, 1), jnp.float32),
  )(deg)


_MB = 1000
_GB = N // _MB


def _mm1_body(x_ref, w_ref, dv_ref, o_ref):
  h = jnp.dot(x_ref[...], w_ref[...], preferred_element_type=jnp.float32)
  o_ref[...] = h * dv_ref[...]


def _mm1_call(x, w1, dinv):
  return pl.pallas_call(
      _mm1_body,
      grid=(2 * _GB,),
      in_specs=[
          pl.BlockSpec((_MB, D), lambda j: (j % _GB, 0)),
          pl.BlockSpec((D, DH), lambda j: (0, j // _GB)),
          pl.BlockSpec((_MB, 1), lambda j: (j % _GB, 0)),
      ],
      out_specs=pl.BlockSpec((_MB, DH), lambda j: (j, 0)),
      out_shape=jax.ShapeDtypeStruct((NC * N, DH), jnp.float32),
  )(x, w1, dinv)


def _mm2_body(sa_ref, sb_ref, ha_ref, hb_ref, dv_ref, b_ref, w_ref, o_ref):
  dv = dv_ref[...]
  b = b_ref[...]
  ua = jnp.maximum(dv * (sa_ref[...] + ha_ref[...]) + b[:, :DH], 0.0)
  ub = jnp.maximum(dv * (sb_ref[...] + hb_ref[...]) + b[:, DH:], 0.0)
  u = jnp.concatenate([ua, ub], axis=1)
  o_ref[...] = dv * jnp.dot(u, w_ref[...], preferred_element_type=jnp.float32)


def _mm2_call(s1, h1p, dinv, b1, w2):
  return pl.pallas_call(
      _mm2_body,
      grid=(2 * _GB,),
      in_specs=[
          pl.BlockSpec((_MB, DH), lambda j: (j % _GB, 0)),
          pl.BlockSpec((_MB, DH), lambda j: (_GB + j % _GB, 0)),
          pl.BlockSpec((_MB, DH), lambda j: (j % _GB, 0)),
          pl.BlockSpec((_MB, DH), lambda j: (_GB + j % _GB, 0)),
          pl.BlockSpec((_MB, 1), lambda j: (j % _GB, 0)),
          pl.BlockSpec((1, D), lambda j: (0, 0)),
          pl.BlockSpec((D, DH), lambda j: (0, j // _GB)),
      ],
      out_specs=pl.BlockSpec((_MB, DH), lambda j: (j, 0)),
      out_shape=jax.ShapeDtypeStruct((NC * N, DH), jnp.float32),
  )(s1, s1, h1p, h1p, dinv, b1, w2)


def _fin_body(sa_ref, sb_ref, ha_ref, hb_ref, dv_ref, b_ref, w_ref,
              t_ref, acc_ref):
  j = pl.program_id(0)
  dv = dv_ref[...]
  b = b_ref[...]
  ua = jnp.maximum(dv * (sa_ref[...] + ha_ref[...]) + b[:, :DH], 0.0)
  ub = jnp.maximum(dv * (sb_ref[...] + hb_ref[...]) + b[:, DH:], 0.0)
  u = jnp.concatenate([ua, ub], axis=1)
  t = dv * jnp.dot(u, w_ref[...], preferred_element_type=jnp.float32)
  t_ref[...] = t

  @pl.when(j == 0)
  def _():
    acc_ref[...] = jnp.zeros((1, 1), jnp.float32)
  acc_ref[...] = acc_ref[...] + jnp.sum(t * dv).reshape(1, 1)


def _fin_call(s2, h2p, dinv, b2, w3):
  return pl.pallas_call(
      _fin_body,
      grid=(_GB,),
      in_specs=[
          pl.BlockSpec((_MB, DH), lambda j: (j, 0)),
          pl.BlockSpec((_MB, DH), lambda j: (_GB + j, 0)),
          pl.BlockSpec((_MB, DH), lambda j: (j, 0)),
          pl.BlockSpec((_MB, DH), lambda j: (_GB + j, 0)),
          pl.BlockSpec((_MB, 1), lambda j: (j, 0)),
          pl.BlockSpec((1, D), lambda j: (0, 0)),
          pl.BlockSpec((D, 1), lambda j: (0, 0)),
      ],
      out_specs=[
          pl.BlockSpec((_MB, 1), lambda j: (j, 0)),
          pl.BlockSpec((1, 1), lambda j: (0, 0)),
      ],
      out_shape=[
          jax.ShapeDtypeStruct((N, 1), jnp.float32),
          jax.ShapeDtypeStruct((1, 1), jnp.float32),
      ],
  )(s2, s2, h2p, h2p, dinv, b2, w3)


@jax.jit
def kernel(x, edge_index, W1, b1, W2, b2, W3, b3):
  E = edge_index.shape[1]
  src = edge_index[0]
  dst = edge_index[1]

  ept = E // NS
  ke = 80
  slab = 25
  assert ept % (slab * ke) == 0
  src_r = src.reshape(NS, ept // (slab * ke), slab, ke)
  dst_r = dst.reshape(NS, ept // (slab * ke), slab, ke)

  deg = jax.ops.segment_sum(jnp.ones((E,), jnp.float32), dst, num_segments=N)
  dinv = _dinv_call(deg.reshape(N, 1))

  def _scat(table):
    out = _scat_call(table, src_r, dst_r, slab, ke)
    return out.reshape(NC, NPAD, DH)[:, :N, :].reshape(NC * N, DH)

  h1p = _mm1_call(x, W1, dinv)
  s1 = _scat(h1p)
  h2p = _mm2_call(s1, h1p, dinv, b1.reshape(1, D), W2)
  s2 = _scat(h2p)
  t, selfacc = _fin_call(s2, h2p, dinv, b2.reshape(1, D), W3)

  tv = t.reshape(N)
  dv = dinv.reshape(N)
  gpart = jnp.sum(tv[src] * dv[dst])
  res = gpart + selfacc[0, 0] + jnp.float32(N) * b3[0]
  return jnp.stack([res])


# speedup vs baseline: 2.7993x; 1.0106x over previous
import jax
import jax.numpy as jnp
from jax import lax

from jax.experimental import pallas as pl
from jax.experimental.pallas import tpu as pltpu
from jax.experimental.pallas import tpu_sc as plsc

N = 10000
D = 256
DH = D // 2
NC = 2
NS = 16
NPTA = 640
NPAD = NS * NPTA
NZC = 64

_MESH = plsc.VectorSubcoreMesh(
    core_axis_name="c", subcore_axis_name="s", num_cores=NC, num_subcores=NS)


def _scat_body(table_hbm, src_hbm, dst_hbm, out_hbm,
               idx_v, dst_v, rows_a, rows_b, zb_v, acc, ga, gb):
  c = lax.axis_index("c")
  s = lax.axis_index("s")

  def _zero(i, _):
    for k in range(DH // 16):
      zb_v[i, pl.ds(k * 16, 16)] = jnp.zeros((16,), jnp.float32)
    return 0
  lax.fori_loop(0, NZC, _zero, 0)
  for ks in range(NS):
    @pl.when(s == ks)
    def _():
      for k in range(NPTA // NZC):
        pltpu.sync_copy(zb_v, acc.at[pl.ds(ks * NPTA + k * NZC, NZC)])
  plsc.subcore_barrier()

  off = c * N
  nslab = src_hbm.shape[1]
  slab = src_hbm.shape[2]

  def _slab(g, _):
    pltpu.sync_copy(src_hbm.at[s].at[g], idx_v)
    pltpu.sync_copy(dst_hbm.at[s].at[g], dst_v)

    def _shift(r, _):
      for k in range(idx_v.shape[1] // 16):
        idx_v[r, pl.ds(k * 16, 16)] = idx_v[r, pl.ds(k * 16, 16)] + off
      return 0
    lax.fori_loop(0, slab, _shift, 0)

    pltpu.async_copy(table_hbm.at[idx_v.at[0]], rows_a, ga)

    def _chunk(j, _):
      @pl.when(j % 2 == 0)
      def _():
        pltpu.make_async_copy(table_hbm.at[idx_v.at[j]], rows_a, ga).wait()

        @pl.when(j + 1 < slab)
        def _():
          pltpu.async_copy(table_hbm.at[idx_v.at[j + 1]], rows_b, gb)
        pltpu.sync_copy(rows_a, acc.at[dst_v.at[j]], add=True)

      @pl.when(j % 2 == 1)
      def _():
        pltpu.make_async_copy(table_hbm.at[idx_v.at[j]], rows_b, gb).wait()

        @pl.when(j + 1 < slab)
        def _():
          pltpu.async_copy(table_hbm.at[idx_v.at[j + 1]], rows_a, ga)
        pltpu.sync_copy(rows_b, acc.at[dst_v.at[j]], add=True)
      return 0
    lax.fori_loop(0, slab, _chunk, 0)
    return 0
  lax.fori_loop(0, nslab, _slab, 0)
  plsc.subcore_barrier()

  nwb = NPTA // NZC
  for ks in range(NS):
    @pl.when(s == ks)
    def _():
      for k in range(nwb):
        pltpu.sync_copy(acc.at[pl.ds(ks * NPTA + k * NZC, NZC)], zb_v)
        pltpu.sync_copy(zb_v, out_hbm.at[c * (NS * nwb) + ks * nwb + k])


def _scat_call(table, src_r, dst_r, slab, ke):
  nwb = NPTA // NZC
  fn = pl.kernel(
      _scat_body,
      out_type=jax.ShapeDtypeStruct((NC * NS * nwb, NZC, DH), jnp.float32),
      mesh=_MESH,
      scratch_types=[
          pltpu.VMEM((slab, ke), jnp.int32),
          pltpu.VMEM((slab, ke), jnp.int32),
          pltpu.VMEM((ke, DH), jnp.float32),
          pltpu.VMEM((ke, DH), jnp.float32),
          pltpu.VMEM((NZC, DH), jnp.float32),
          pltpu.VMEM_SHARED((NPAD, DH), jnp.float32),
          pltpu.SemaphoreType.DMA,
          pltpu.SemaphoreType.DMA,
      ],
  )
  return fn(table, src_r, dst_r)


def _dinv_body(deg_ref, di_ref):
  di_ref[...] = lax.rsqrt(jnp.maximum(deg_ref[...] + 1.0, 1.0))


def _dinv_call(deg):
  return pl.pallas_call(
      _dinv_body,
      out_shape=jax.ShapeDtypeStruct((N, 1), jnp.float32),
  )(deg)


_MB = 1000
_GB = N // _MB


def _mm1_body(x_ref, w_ref, dv_ref, o_ref):
  h = jnp.dot(x_ref[...], w_ref[...], preferred_element_type=jnp.float32)
  o_ref[...] = h * dv_ref[...]


def _mm1_call(x, w1, dinv):
  return pl.pallas_call(
      _mm1_body,
      grid=(2 * _GB,),
      in_specs=[
          pl.BlockSpec((_MB, D), lambda j: (j % _GB, 0)),
          pl.BlockSpec((D, DH), lambda j: (0, j // _GB)),
          pl.BlockSpec((_MB, 1), lambda j: (j % _GB, 0)),
      ],
      out_specs=pl.BlockSpec((_MB, DH), lambda j: (j, 0)),
      out_shape=jax.ShapeDtypeStruct((NC * N, DH), jnp.float32),
  )(x, w1, dinv)


def _mm2_body(sa_ref, sb_ref, ha_ref, hb_ref, dv_ref, b_ref, w_ref, o_ref):
  dv = dv_ref[...]
  b = b_ref[...]
  ua = jnp.maximum(dv * (sa_ref[...] + ha_ref[...]) + b[:, :DH], 0.0)
  ub = jnp.maximum(dv * (sb_ref[...] + hb_ref[...]) + b[:, DH:], 0.0)
  u = jnp.concatenate([ua, ub], axis=1)
  o_ref[...] = dv * jnp.dot(u, w_ref[...], preferred_element_type=jnp.float32)


def _mm2_call(s1, h1p, dinv, b1, w2):
  return pl.pallas_call(
      _mm2_body,
      grid=(2 * _GB,),
      in_specs=[
          pl.BlockSpec((_MB, DH), lambda j: (j % _GB, 0)),
          pl.BlockSpec((_MB, DH), lambda j: (_GB + j % _GB, 0)),
          pl.BlockSpec((_MB, DH), lambda j: (j % _GB, 0)),
          pl.BlockSpec((_MB, DH), lambda j: (_GB + j % _GB, 0)),
          pl.BlockSpec((_MB, 1), lambda j: (j % _GB, 0)),
          pl.BlockSpec((1, D), lambda j: (0, 0)),
          pl.BlockSpec((D, DH), lambda j: (0, j // _GB)),
      ],
      out_specs=pl.BlockSpec((_MB, DH), lambda j: (j, 0)),
      out_shape=jax.ShapeDtypeStruct((NC * N, DH), jnp.float32),
  )(s1, s1, h1p, h1p, dinv, b1, w2)


def _fin_body(sa_ref, sb_ref, ha_ref, hb_ref, dv_ref, b_ref, w_ref,
              t_ref, acc_ref):
  j = pl.program_id(0)
  dv = dv_ref[...]
  b = b_ref[...]
  ua = jnp.maximum(dv * (sa_ref[...] + ha_ref[...]) + b[:, :DH], 0.0)
  ub = jnp.maximum(dv * (sb_ref[...] + hb_ref[...]) + b[:, DH:], 0.0)
  u = jnp.concatenate([ua, ub], axis=1)
  t = dv * jnp.dot(u, w_ref[...], preferred_element_type=jnp.float32)
  t_ref[...] = t

  @pl.when(j == 0)
  def _():
    acc_ref[...] = jnp.zeros((1, 1), jnp.float32)
  acc_ref[...] = acc_ref[...] + jnp.sum(t * dv).reshape(1, 1)


def _fin_call(s2, h2p, dinv, b2, w3):
  return pl.pallas_call(
      _fin_body,
      grid=(_GB,),
      in_specs=[
          pl.BlockSpec((_MB, DH), lambda j: (j, 0)),
          pl.BlockSpec((_MB, DH), lambda j: (_GB + j, 0)),
          pl.BlockSpec((_MB, DH), lambda j: (j, 0)),
          pl.BlockSpec((_MB, DH), lambda j: (_GB + j, 0)),
          pl.BlockSpec((_MB, 1), lambda j: (j, 0)),
          pl.BlockSpec((1, D), lambda j: (0, 0)),
          pl.BlockSpec((D, 1), lambda j: (0, 0)),
      ],
      out_specs=[
          pl.BlockSpec((_MB, 1), lambda j: (j, 0)),
          pl.BlockSpec((1, 1), lambda j: (0, 0)),
      ],
      out_shape=[
          jax.ShapeDtypeStruct((N, 1), jnp.float32),
          jax.ShapeDtypeStruct((1, 1), jnp.float32),
      ],
  )(s2, s2, h2p, h2p, dinv, b2, w3)


@jax.jit
def kernel(x, edge_index, W1, b1, W2, b2, W3, b3):
  E = edge_index.shape[1]
  src = edge_index[0]
  dst = edge_index[1]

  ept = E // NS
  ke = 80
  slab = 5
  assert ept % (slab * ke) == 0
  src_r = src.reshape(NS, ept // (slab * ke), slab, ke)
  dst_r = dst.reshape(NS, ept // (slab * ke), slab, ke)

  deg = jax.ops.segment_sum(jnp.ones((E,), jnp.float32), dst, num_segments=N)
  dinv = _dinv_call(deg.reshape(N, 1))

  def _scat(table):
    out = _scat_call(table, src_r, dst_r, slab, ke)
    return out.reshape(NC, NPAD, DH)[:, :N, :].reshape(NC * N, DH)

  h1p = _mm1_call(x, W1, dinv)
  s1 = _scat(h1p)
  h2p = _mm2_call(s1, h1p, dinv, b1.reshape(1, D), W2)
  s2 = _scat(h2p)
  t, selfacc = _fin_call(s2, h2p, dinv, b2.reshape(1, D), W3)

  tv = t.reshape(N)
  dv = dinv.reshape(N)
  gpart = jnp.sum(tv[src] * dv[dst])
  res = gpart + selfacc[0, 0] + jnp.float32(N) * b3[0]
  return jnp.stack([res])
